# baseline (speedup 1.0000x reference)
import jax
import jax.numpy as jnp
from jax.experimental import pallas as pl
from jax.experimental.pallas import tpu as pltpu

HN, DD, N1, N2, Q1V, Q2V, MV = 8, 32, 128, 128, 21, 21, 1024
LAMBD = 0.001
BP = 128


def _fused_kernel(q_ref, k_ref, z1_ref, z2_ref, w_ref, v2_ref, v_ref,
                  out_ref):
    sfs = []
    for h in range(HN):
        e = jax.lax.dot_general(
            q_ref[h], k_ref[h], (((0,), (0,)), ((), ())),
            preferred_element_type=jnp.float32)
        mx = jnp.max(e, axis=1, keepdims=True)
        ex = jnp.exp(e - mx)
        sfs.append(ex / jnp.sum(ex, axis=1, keepdims=True))

    reg = jnp.float32(0.0)
    for h in range(HN):
        for k in range(h, HN):
            mult = jnp.float32(2.0 if k != h else 1.0)
            s_hk = jnp.sum(sfs[h] * sfs[k])
            v_hk = jnp.sum(v_ref[h] * v_ref[k])
            reg = reg + mult * s_hk * v_hk
    reg = jnp.float32(LAMBD) * reg

    z1 = z1_ref[...]
    z2 = z2_ref[...]

    ene2 = jnp.zeros((N1, MV), jnp.float32)
    se2 = jnp.zeros((N1, MV), jnp.float32)
    for a in range(Q1V):
        lt = None
        for h in range(HN):
            row = jnp.broadcast_to(v2_ref[h * 32 + a:h * 32 + a + 1, :],
                                   (N2, BP))
            dh = jnp.take_along_axis(row, z2, axis=1,
                                     mode="promise_in_bounds")
            p = jax.lax.dot_general(
                sfs[h], dh, (((1,), (0,)), ((), ())),
                preferred_element_type=jnp.float32)
            lt = p if lt is None else lt + p
        ene2 = ene2 + jnp.where(z1 == a, lt, 0.0)
        se2 = se2 + jnp.exp(lt)

    ene = jnp.sum(ene2, axis=0, keepdims=True)
    s = jnp.sum(se2, axis=0, keepdims=True) + jnp.float32(N1 - Q1V)
    logzf = jnp.log(s)

    part = jnp.sum(w_ref[...] * (ene - logzf))
    out_ref[...] = (reg - part).reshape(1, 1)


def kernel(Z1, Z2, weights, Q, K, V):
    Z1 = Z1.astype(jnp.int32)
    Z2 = Z2.astype(jnp.int32)
    w = weights.astype(jnp.float32).reshape(1, MV)
    v2 = jnp.pad(V, ((0, 0), (0, 32 - Q1V), (0, BP - Q2V))).reshape(HN * 32, BP)

    out = pl.pallas_call(
        _fused_kernel,
        in_specs=[
            pl.BlockSpec((HN, DD, N1), lambda: (0, 0, 0)),
            pl.BlockSpec((HN, DD, N2), lambda: (0, 0, 0)),
            pl.BlockSpec((N1, MV), lambda: (0, 0)),
            pl.BlockSpec((N2, MV), lambda: (0, 0)),
            pl.BlockSpec((1, MV), lambda: (0, 0)),
            pl.BlockSpec((HN * 32, BP), lambda: (0, 0)),
            pl.BlockSpec((HN, Q1V, Q2V), lambda: (0, 0, 0)),
        ],
        out_specs=pl.BlockSpec((1, 1), lambda: (0, 0)),
        out_shape=jax.ShapeDtypeStruct((1, 1), jnp.float32),
    )(Q, K, Z1, Z2, w, v2, V)
    return out[0, 0]


# speedup vs baseline: 1.1713x; 1.1713x over previous
import jax
import jax.numpy as jnp
from jax.experimental import pallas as pl
from jax.experimental.pallas import tpu as pltpu

HN, DD, N1, N2, Q1V, Q2V, MV = 8, 32, 128, 128, 21, 21, 1024
LAMBD = 0.001
BP = 128
MB = 256
NB = MV // MB


def _fused_kernel(q_ref, k_ref, z1_ref, z2_ref, w_ref, v2_ref, v_ref,
                  out_ref, sfw_ref, reg_ref):
    b = pl.program_id(0)

    @pl.when(b == 0)
    def _():
        sfs = []
        for h in range(HN):
            e = jax.lax.dot_general(
                q_ref[h], k_ref[h], (((0,), (0,)), ((), ())),
                preferred_element_type=jnp.float32)
            mx = jnp.max(e, axis=1, keepdims=True)
            ex = jnp.exp(e - mx)
            sf = ex / jnp.sum(ex, axis=1, keepdims=True)
            sfw_ref[:, h * N2:(h + 1) * N2] = sf
            sfs.append(sf)
        reg = jnp.float32(0.0)
        for h in range(HN):
            for k in range(h, HN):
                mult = jnp.float32(2.0 if k != h else 1.0)
                s_hk = jnp.sum(sfs[h] * sfs[k])
                v_hk = jnp.sum(v_ref[h] * v_ref[k])
                reg = reg + mult * s_hk * v_hk
        reg_ref[...] = (jnp.float32(LAMBD) * reg).reshape(1, 1)

    z1 = z1_ref[...]
    z2 = z2_ref[...]
    sfw = sfw_ref[...]

    ene2 = jnp.zeros((N1, MB), jnp.float32)
    se2 = jnp.zeros((N1, MB), jnp.float32)
    for a in range(Q1V):
        lt = None
        for h in range(HN):
            row = jnp.broadcast_to(v2_ref[h * 32 + a:h * 32 + a + 1, :],
                                   (N2, BP))
            dh = jnp.take_along_axis(row, z2, axis=1,
                                     mode="promise_in_bounds")
            p = jax.lax.dot_general(
                sfw[:, h * N2:(h + 1) * N2], dh, (((1,), (0,)), ((), ())),
                preferred_element_type=jnp.float32)
            lt = p if lt is None else lt + p
        ene2 = ene2 + jnp.where(z1 == a, lt, 0.0)
        se2 = se2 + jnp.exp(lt)

    ene = jnp.sum(ene2, axis=0, keepdims=True)
    s = jnp.sum(se2, axis=0, keepdims=True) + jnp.float32(N1 - Q1V)
    logzf = jnp.log(s)

    part = jnp.sum(w_ref[...] * (ene - logzf))

    @pl.when(b == 0)
    def _():
        out_ref[...] = reg_ref[...] - part.reshape(1, 1)

    @pl.when(b != 0)
    def _():
        out_ref[...] = out_ref[...] - part.reshape(1, 1)


def kernel(Z1, Z2, weights, Q, K, V):
    Z1 = Z1.astype(jnp.int32)
    Z2 = Z2.astype(jnp.int32)
    w = weights.astype(jnp.float32).reshape(1, MV)
    v2 = jnp.pad(V, ((0, 0), (0, 32 - Q1V), (0, BP - Q2V))).reshape(HN * 32, BP)

    out = pl.pallas_call(
        _fused_kernel,
        grid=(NB,),
        in_specs=[
            pl.BlockSpec((HN, DD, N1), lambda b: (0, 0, 0)),
            pl.BlockSpec((HN, DD, N2), lambda b: (0, 0, 0)),
            pl.BlockSpec((N1, MB), lambda b: (0, b)),
            pl.BlockSpec((N2, MB), lambda b: (0, b)),
            pl.BlockSpec((1, MB), lambda b: (0, b)),
            pl.BlockSpec((HN * 32, BP), lambda b: (0, 0)),
            pl.BlockSpec((HN, Q1V, Q2V), lambda b: (0, 0, 0)),
        ],
        out_specs=pl.BlockSpec((1, 1), lambda b: (0, 0)),
        out_shape=jax.ShapeDtypeStruct((1, 1), jnp.float32),
        scratch_shapes=[
            pltpu.VMEM((N1, HN * N2), jnp.float32),
            pltpu.VMEM((1, 1), jnp.float32),
        ],
    )(Q, K, Z1, Z2, w, v2, V)
    return out[0, 0]


# speedup vs baseline: 1.1760x; 1.0040x over previous
import jax
import jax.numpy as jnp
from jax.experimental import pallas as pl
from jax.experimental.pallas import tpu as pltpu

HN, DD, N1, N2, Q1V, Q2V, MV = 8, 32, 128, 128, 21, 21, 1024
LAMBD = 0.001
BP = 128
MB = 256
NB = MV // MB


def _fused_kernel(q_ref, k_ref, z1_ref, z2_ref, w_ref, v_ref,
                  out_ref, sfw_ref, reg_ref):
    b = pl.program_id(0)

    @pl.when(b == 0)
    def _():
        sfs = []
        for h in range(HN):
            e = jax.lax.dot_general(
                q_ref[h], k_ref[h], (((0,), (0,)), ((), ())),
                preferred_element_type=jnp.float32)
            mx = jnp.max(e, axis=1, keepdims=True)
            ex = jnp.exp(e - mx)
            sf = ex / jnp.sum(ex, axis=1, keepdims=True)
            sfw_ref[:, h * N2:(h + 1) * N2] = sf
            sfs.append(sf)
        reg = jnp.float32(0.0)
        for h in range(HN):
            for k in range(h, HN):
                mult = jnp.float32(2.0 if k != h else 1.0)
                s_hk = jnp.sum(sfs[h] * sfs[k])
                v_hk = jnp.sum(v_ref[h] * v_ref[k])
                reg = reg + mult * s_hk * v_hk
        reg_ref[...] = (jnp.float32(LAMBD) * reg).reshape(1, 1)

    z1 = z1_ref[...]
    z2 = z2_ref[...]
    sfw = sfw_ref[...]

    ene2 = jnp.zeros((N1, MB), jnp.float32)
    se2 = jnp.zeros((N1, MB), jnp.float32)
    for a in range(Q1V):
        lt = None
        for h in range(HN):
            row = jnp.broadcast_to(v_ref[h, a:a + 1, :], (N2, Q2V))
            dh = jnp.take_along_axis(row, z2, axis=1,
                                     mode="promise_in_bounds")
            p = jax.lax.dot_general(
                sfw[:, h * N2:(h + 1) * N2], dh, (((1,), (0,)), ((), ())),
                preferred_element_type=jnp.float32)
            lt = p if lt is None else lt + p
        ene2 = ene2 + jnp.where(z1 == a, lt, 0.0)
        se2 = se2 + jnp.exp(lt)

    ene = jnp.sum(ene2, axis=0, keepdims=True)
    s = jnp.sum(se2, axis=0, keepdims=True) + jnp.float32(N1 - Q1V)
    logzf = jnp.log(s)

    part = jnp.sum(w_ref[...] * (ene - logzf))

    @pl.when(b == 0)
    def _():
        out_ref[...] = reg_ref[...] - part.reshape(1, 1)

    @pl.when(b != 0)
    def _():
        out_ref[...] = out_ref[...] - part.reshape(1, 1)


def kernel(Z1, Z2, weights, Q, K, V):
    Z1 = Z1.astype(jnp.int32)
    Z2 = Z2.astype(jnp.int32)
    w = weights.astype(jnp.float32).reshape(1, MV)

    out = pl.pallas_call(
        _fused_kernel,
        grid=(NB,),
        in_specs=[
            pl.BlockSpec((HN, DD, N1), lambda b: (0, 0, 0)),
            pl.BlockSpec((HN, DD, N2), lambda b: (0, 0, 0)),
            pl.BlockSpec((N1, MB), lambda b: (0, b)),
            pl.BlockSpec((N2, MB), lambda b: (0, b)),
            pl.BlockSpec((1, MB), lambda b: (0, b)),
            pl.BlockSpec((HN, Q1V, Q2V), lambda b: (0, 0, 0)),
        ],
        out_specs=pl.BlockSpec((1, 1), lambda b: (0, 0)),
        out_shape=jax.ShapeDtypeStruct((1, 1), jnp.float32),
        scratch_shapes=[
            pltpu.VMEM((N1, HN * N2), jnp.float32),
            pltpu.VMEM((1, 1), jnp.float32),
        ],
    )(Q, K, Z1, Z2, w, V)
    return out[0, 0]
